# baseline (speedup 1.0000x reference)
import functools

import jax
import jax.numpy as jnp
from jax import lax
from jax.experimental import pallas as pl
from jax.experimental.pallas import tpu as pltpu
from jax.experimental.pallas import tpu_sc as plsc

N = 50000
K = 16
CIN = 4
C = 64
H = 64
NBLK = 4

NW = 32
NP = 51200
M = N * K
MPAD = NP * K
CE = 2560
PW_E = MPAD // 16
CH = 80
PW = NP // 16
KCH = K * CH

@functools.lru_cache(maxsize=None)
def _sc_kernels():
    mesh = plsc.VectorSubcoreMesh(
        core_axis_name="c", subcore_axis_name="s", num_cores=1)

    @functools.partial(
        pl.kernel,
        out_type=jax.ShapeDtypeStruct((MPAD, 8), jnp.float32),
        mesh=mesh,
        scratch_types=[
            pltpu.VMEM((2, CE), jnp.int32),
            pltpu.VMEM((2, CE, 8), jnp.float32),
            pltpu.SemaphoreType.DMA,
            pltpu.SemaphoreType.DMA,
            pltpu.SemaphoreType.DMA,
        ],
        compiler_params=pltpu.CompilerParams(use_tc_tiling_on_sc=False),
    )
    def sc_head_gather(idx_hbm, pf_hbm, out_hbm, idx_v, buf_v, gsem, isem, osem):
        base = lax.axis_index("s") * PW_E
        nch = PW_E // CE

        def islice(c):
            return idx_hbm.at[pl.ds(base + c * CE, CE)]

        def oslice(c):
            return out_hbm.at[pl.ds(base + c * CE, CE)]

        pltpu.sync_copy(islice(0), idx_v.at[0])
        pltpu.async_copy(pf_hbm.at[idx_v.at[0]], buf_v.at[0], gsem)
        pltpu.async_copy(islice(1), idx_v.at[1], isem)

        @pl.loop(0, nch)
        def _chunk(c):
            p = c % 2
            pltpu.make_async_copy(pf_hbm.at[idx_v.at[p]], buf_v.at[p],
                                  gsem).wait()

            @pl.when(c >= 1)
            def _():
                pltpu.make_async_copy(buf_v.at[1 - p], oslice(c - 1),
                                      osem).wait()

            @pl.when(c + 1 < nch)
            def _():
                pltpu.make_async_copy(islice(c + 1), idx_v.at[1 - p],
                                      isem).wait()
                pltpu.async_copy(pf_hbm.at[idx_v.at[1 - p]], buf_v.at[1 - p],
                                 gsem)

            @pl.when(c + 2 < nch)
            def _():
                pltpu.async_copy(islice(c + 2), idx_v.at[p], isem)

            pltpu.async_copy(buf_v.at[p], oslice(c), osem)

        pltpu.make_async_copy(buf_v.at[(nch - 1) % 2], oslice(nch - 1),
                              osem).wait()

    @functools.partial(
        pl.kernel,
        out_type=jax.ShapeDtypeStruct((NP, C), jnp.bfloat16),
        mesh=mesh,
        scratch_types=[
            pltpu.VMEM((2, KCH), jnp.int32),
            pltpu.VMEM((2, KCH, C), jnp.bfloat16),
            pltpu.VMEM((2, CH, C), jnp.bfloat16),
            pltpu.SemaphoreType.DMA,
            pltpu.SemaphoreType.DMA,
            pltpu.SemaphoreType.DMA,
        ],
        compiler_params=pltpu.CompilerParams(use_tc_tiling_on_sc=False),
    )
    def sc_gathermax(idx_hbm, h_hbm, out_hbm, idx_v, buf_v, gm_v,
                     gsem, isem, osem):
        nbase = lax.axis_index("s") * PW
        nch = PW // CH

        def islice(c):
            return idx_hbm.at[pl.ds((nbase + c * CH) * K, KCH)]

        def oslice(c):
            return out_hbm.at[pl.ds(nbase + c * CH, CH)]

        pltpu.sync_copy(islice(0), idx_v.at[0])
        pltpu.async_copy(h_hbm.at[idx_v.at[0]], buf_v.at[0], gsem)
        pltpu.async_copy(islice(1), idx_v.at[1], isem)

        @pl.loop(0, nch)
        def _chunk(c):
            p = c % 2
            pltpu.make_async_copy(h_hbm.at[idx_v.at[p]], buf_v.at[p],
                                  gsem).wait()

            @pl.when(c + 1 < nch)
            def _():
                pltpu.make_async_copy(islice(c + 1), idx_v.at[1 - p],
                                      isem).wait()
                pltpu.async_copy(h_hbm.at[idx_v.at[1 - p]], buf_v.at[1 - p],
                                 gsem)

            @pl.when(c + 2 < nch)
            def _():
                pltpu.async_copy(islice(c + 2), idx_v.at[p], isem)

            @pl.when(c >= 1)
            def _():
                pltpu.make_async_copy(gm_v.at[1 - p], oslice(c - 1),
                                      osem).wait()

            def body(r, _):
                for cc in range(C // 32):
                    sl = pl.ds(cc * 32, 32)
                    v = buf_v[p, r * K, sl]
                    for k in range(1, K):
                        v = jnp.maximum(v, buf_v[p, r * K + k, sl])
                    gm_v[p, r, sl] = v
                return 0

            lax.fori_loop(0, CH, body, 0)
            pltpu.async_copy(gm_v.at[p], oslice(c), osem)

        pltpu.make_async_copy(gm_v.at[(nch - 1) % 2], oslice(nch - 1),
                              osem).wait()

    return sc_head_gather, sc_gathermax


def _sc_head_gather(idx_flat, pf):
    return _sc_kernels()[0](idx_flat, pf)


def _sc_gathermax(idx_flat, h):
    return _sc_kernels()[1](idx_flat, h)


BN = 2048
BE4 = BN * 4


def _tc_head_body(xe_ref, p32_ref, w1_ref, g1_ref, b1_ref, w2_ref, g2_ref,
                  b2_ref, w3_ref, gg_ref, gb_ref, fe_ref):
    a = jnp.dot(xe_ref[...], w1_ref[...],
                preferred_element_type=jnp.float32)
    b = jnp.dot(p32_ref[...], w1_ref[...],
                preferred_element_type=jnp.float32)
    b4 = jnp.broadcast_to(b[:, None, :], (BN, 4, 128)).reshape(BE4, 128)
    y = jax.nn.gelu((a - b4) * g1_ref[...] + b1_ref[...])
    y = jnp.dot(y, w2_ref[...], preferred_element_type=jnp.float32)
    y = jax.nn.gelu(y * g2_ref[...] + b2_ref[...])
    y = jnp.dot(y, w3_ref[...], preferred_element_type=jnp.float32)
    y = jnp.max(y.reshape(BN, 4, 256), axis=1)
    m = jnp.maximum(jnp.maximum(y[:, 0:64], y[:, 64:128]),
                    jnp.maximum(y[:, 128:192], y[:, 192:256]))
    fe_ref[...] = m * gg_ref[...] + gb_ref[...]


def _tc_head(xe4, p32, w1bd, g1t, b1t, w2bd, g2t, b2t, w3bd, gg, gb):
    nblocks = MPAD // 4 // BE4
    full = lambda shape: pl.BlockSpec(shape, lambda i: (0, 0))
    return pl.pallas_call(
        _tc_head_body,
        grid=(nblocks,),
        in_specs=[
            pl.BlockSpec((BE4, 32), lambda i: (i, 0)),
            pl.BlockSpec((BN, 32), lambda i: (i, 0)),
            full((32, 128)), full((1, 128)), full((1, 128)),
            full((128, 256)), full((1, 256)), full((1, 256)),
            full((256, 256)), full((1, 64)), full((1, 64)),
        ],
        out_specs=pl.BlockSpec((BN, C), lambda i: (i, 0)),
        out_shape=jax.ShapeDtypeStruct((NP, C), jnp.float32),
    )(xe4, p32, w1bd, g1t, b1t, w2bd, g2t, b2t, w3bd, gg, gb)


BM = 2048


def _tc_mlp_body(x_ref, g_ref, rg_ref, rb_ref, wa_ref, wb_ref,
                 fe_ref, h_ref):
    fe = x_ref[...] + g_ref[...].astype(jnp.float32)
    fe_ref[...] = fe
    x = fe * rg_ref[...] + rb_ref[...]
    y = jnp.dot(x, wa_ref[...], preferred_element_type=jnp.float32)
    y = jax.nn.gelu(y)
    h = jnp.dot(y, wb_ref[...], preferred_element_type=jnp.float32)
    h_ref[...] = h.astype(jnp.bfloat16)


def _tc_mlp(fe, g, rg1, rb1, wa, wb):
    nblocks = NP // BM
    full = lambda shape: pl.BlockSpec(shape, lambda i: (0, 0))
    return pl.pallas_call(
        _tc_mlp_body,
        grid=(nblocks,),
        in_specs=[
            pl.BlockSpec((BM, C), lambda i: (i, 0)),
            pl.BlockSpec((BM, C), lambda i: (i, 0)),
            full((1, C)), full((1, C)),
            full((C, 128)), full((128, C)),
        ],
        out_specs=[
            pl.BlockSpec((BM, C), lambda i: (i, 0)),
            pl.BlockSpec((BM, C), lambda i: (i, 0)),
        ],
        out_shape=[
            jax.ShapeDtypeStruct((NP, C), jnp.float32),
            jax.ShapeDtypeStruct((NP, C), jnp.bfloat16),
        ],
    )(fe, g, rg1, rb1, wa, wb)


def _tc_add_body(x_ref, g_ref, o_ref):
    o_ref[...] = x_ref[...] + g_ref[...].astype(jnp.float32)


def _tc_add(fe, g):
    nblocks = NP // BM
    return pl.pallas_call(
        _tc_add_body,
        grid=(nblocks,),
        in_specs=[
            pl.BlockSpec((BM, C), lambda i: (i, 0)),
            pl.BlockSpec((BM, C), lambda i: (i, 0)),
        ],
        out_specs=pl.BlockSpec((BM, C), lambda i: (i, 0)),
        out_shape=jax.ShapeDtypeStruct((NP, C), jnp.float32),
    )(fe, g)


def _block_diag(*ms):
    rows = sum(m.shape[0] for m in ms)
    cols = sum(m.shape[1] for m in ms)
    out = jnp.zeros((rows, cols), ms[0].dtype)
    r = c = 0
    for m in ms:
        out = out.at[r:r + m.shape[0], c:c + m.shape[1]].set(m)
        r += m.shape[0]
        c += m.shape[1]
    return out


def kernel(p, p_gs, f, group_idx, W1, g1, b1, W2, g2, b2, W3,
           gnb_g, gnb_b, Wa, Wb, rg, rb):
    del p_gs
    idx32 = group_idx.astype(jnp.int32)
    idx_flat = jnp.concatenate(
        [idx32.reshape(-1), jnp.zeros((MPAD - M,), jnp.int32)])

    pf = jnp.concatenate([p, f, jnp.zeros((N, 1), jnp.float32)], axis=1)
    ps8 = jnp.concatenate([p, jnp.zeros((N, 5), jnp.float32)], axis=1)
    ps8 = jnp.concatenate([ps8, jnp.zeros((NP - N, 8), jnp.float32)], axis=0)
    p32 = jnp.tile(ps8, (1, 4))

    w1p = jnp.concatenate([W1, jnp.zeros((1, 32), jnp.float32)], axis=0)
    w1bd = _block_diag(w1p, w1p, w1p, w1p)
    g1t = jnp.tile(g1, 4)[None, :]
    b1t = jnp.tile(b1, 4)[None, :]
    w2bd = _block_diag(W2, W2, W2, W2)
    g2t = jnp.tile(g2, 4)[None, :]
    b2t = jnp.tile(b2, 4)[None, :]
    w3bd = _block_diag(W3, W3, W3, W3)

    xe = _sc_head_gather(idx_flat, pf)
    fe = _tc_head(xe.reshape(MPAD // 4, 32), p32,
                  w1bd, g1t, b1t, w2bd, g2t, b2t, w3bd,
                  gnb_g[None, :], gnb_b[None, :])

    g = jnp.zeros((NP, C), jnp.bfloat16)
    for i in range(NBLK):
        fe, h = _tc_mlp(fe, g, rg[i][None, :], rb[i][None, :], Wa[i], Wb[i])
        g = _sc_gathermax(idx_flat, h)

    return _tc_add(fe, g)[:N]


# speedup vs baseline: 1.0091x; 1.0091x over previous
import functools

import jax
import jax.numpy as jnp
from jax import lax
from jax.experimental import pallas as pl
from jax.experimental.pallas import tpu as pltpu
from jax.experimental.pallas import tpu_sc as plsc

N = 50000
K = 16
CIN = 4
C = 64
H = 64
NBLK = 4

NW = 32
NP = 51200
M = N * K
MPAD = NP * K
CE = 2560
E0 = 51200
E1 = 0
CH = 80
F0 = 3200
F1 = 0
KCH = K * CH

@functools.lru_cache(maxsize=None)
def _sc_kernels():
    mesh = plsc.VectorSubcoreMesh(core_axis_name="c", subcore_axis_name="s")

    @functools.partial(
        pl.kernel,
        out_type=jax.ShapeDtypeStruct((MPAD, 8), jnp.float32),
        mesh=mesh,
        scratch_types=[
            pltpu.VMEM((2, CE), jnp.int32),
            pltpu.VMEM((2, CE, 8), jnp.float32),
            pltpu.SemaphoreType.DMA,
            pltpu.SemaphoreType.DMA,
            pltpu.SemaphoreType.DMA,
        ],
        compiler_params=pltpu.CompilerParams(use_tc_tiling_on_sc=False),
    )
    def sc_head_gather(idx_hbm, pf_hbm, out_hbm, idx_v, buf_v, gsem, isem, osem):
        cid = lax.axis_index("c")
        sid = lax.axis_index("s")
        base = jnp.where(cid == 0, sid * E0, 0)
        nch = jnp.where(cid == 0, E0 // CE, E1 // CE)

        def islice(c):
            return idx_hbm.at[pl.ds(base + c * CE, CE)]

        def oslice(c):
            return out_hbm.at[pl.ds(base + c * CE, CE)]

        @pl.when(nch >= 1)
        def _():
            pltpu.sync_copy(islice(0), idx_v.at[0])
            pltpu.async_copy(pf_hbm.at[idx_v.at[0]], buf_v.at[0], gsem)

        @pl.when(nch >= 2)
        def _():
            pltpu.async_copy(islice(1), idx_v.at[1], isem)

        @pl.loop(0, nch)
        def _chunk(c):
            p = c % 2
            pltpu.make_async_copy(pf_hbm.at[idx_v.at[p]], buf_v.at[p],
                                  gsem).wait()

            @pl.when(c >= 1)
            def _():
                pltpu.make_async_copy(buf_v.at[1 - p], oslice(c - 1),
                                      osem).wait()

            @pl.when(c + 1 < nch)
            def _():
                pltpu.make_async_copy(islice(c + 1), idx_v.at[1 - p],
                                      isem).wait()
                pltpu.async_copy(pf_hbm.at[idx_v.at[1 - p]], buf_v.at[1 - p],
                                 gsem)

            @pl.when(c + 2 < nch)
            def _():
                pltpu.async_copy(islice(c + 2), idx_v.at[p], isem)

            pltpu.async_copy(buf_v.at[p], oslice(c), osem)

        @pl.when(nch >= 1)
        def _():
            pltpu.make_async_copy(buf_v.at[(nch - 1) % 2], oslice(nch - 1),
                                  osem).wait()

    @functools.partial(
        pl.kernel,
        out_type=jax.ShapeDtypeStruct((NP, C), jnp.bfloat16),
        mesh=mesh,
        scratch_types=[
            pltpu.VMEM((2, KCH), jnp.int32),
            pltpu.VMEM((2, KCH, C), jnp.bfloat16),
            pltpu.VMEM((2, CH, C), jnp.bfloat16),
            pltpu.SemaphoreType.DMA,
            pltpu.SemaphoreType.DMA,
            pltpu.SemaphoreType.DMA,
        ],
        compiler_params=pltpu.CompilerParams(use_tc_tiling_on_sc=False),
    )
    def sc_gathermax(idx_hbm, h_hbm, out_hbm, idx_v, buf_v, gm_v,
                     gsem, isem, osem):
        cid = lax.axis_index("c")
        sid = lax.axis_index("s")
        nbase = jnp.where(cid == 0, sid * F0, 0)
        nch = jnp.where(cid == 0, F0 // CH, F1 // CH)

        def islice(c):
            return idx_hbm.at[pl.ds((nbase + c * CH) * K, KCH)]

        def oslice(c):
            return out_hbm.at[pl.ds(nbase + c * CH, CH)]

        @pl.when(nch >= 1)
        def _():
            pltpu.sync_copy(islice(0), idx_v.at[0])
            pltpu.async_copy(h_hbm.at[idx_v.at[0]], buf_v.at[0], gsem)

        @pl.when(nch >= 2)
        def _():
            pltpu.async_copy(islice(1), idx_v.at[1], isem)

        @pl.loop(0, nch)
        def _chunk(c):
            p = c % 2
            pltpu.make_async_copy(h_hbm.at[idx_v.at[p]], buf_v.at[p],
                                  gsem).wait()

            @pl.when(c + 1 < nch)
            def _():
                pltpu.make_async_copy(islice(c + 1), idx_v.at[1 - p],
                                      isem).wait()
                pltpu.async_copy(h_hbm.at[idx_v.at[1 - p]], buf_v.at[1 - p],
                                 gsem)

            @pl.when(c + 2 < nch)
            def _():
                pltpu.async_copy(islice(c + 2), idx_v.at[p], isem)

            @pl.when(c >= 1)
            def _():
                pltpu.make_async_copy(gm_v.at[1 - p], oslice(c - 1),
                                      osem).wait()

            def body(r, _):
                for cc in range(C // 32):
                    sl = pl.ds(cc * 32, 32)
                    v = buf_v[p, r * K, sl]
                    for k in range(1, K):
                        v = jnp.maximum(v, buf_v[p, r * K + k, sl])
                    gm_v[p, r, sl] = v
                return 0

            lax.fori_loop(0, CH, body, 0)
            pltpu.async_copy(gm_v.at[p], oslice(c), osem)

        @pl.when(nch >= 1)
        def _():
            pltpu.make_async_copy(gm_v.at[(nch - 1) % 2], oslice(nch - 1),
                                  osem).wait()

    return sc_head_gather, sc_gathermax


def _sc_head_gather(idx_flat, pf):
    return _sc_kernels()[0](idx_flat, pf)


def _sc_gathermax(idx_flat, h):
    return _sc_kernels()[1](idx_flat, h)


BN = 2048
BE4 = BN * 4


def _tc_head_body(xe_ref, p32_ref, w1_ref, g1_ref, b1_ref, w2_ref, g2_ref,
                  b2_ref, w3_ref, gg_ref, gb_ref, fe_ref):
    a = jnp.dot(xe_ref[...], w1_ref[...],
                preferred_element_type=jnp.float32)
    b = jnp.dot(p32_ref[...], w1_ref[...],
                preferred_element_type=jnp.float32)
    b4 = jnp.broadcast_to(b[:, None, :], (BN, 4, 128)).reshape(BE4, 128)
    y = jax.nn.gelu((a - b4) * g1_ref[...] + b1_ref[...])
    y = jnp.dot(y, w2_ref[...], preferred_element_type=jnp.float32)
    y = jax.nn.gelu(y * g2_ref[...] + b2_ref[...])
    y = jnp.dot(y, w3_ref[...], preferred_element_type=jnp.float32)
    y = jnp.max(y.reshape(BN, 4, 256), axis=1)
    m = jnp.maximum(jnp.maximum(y[:, 0:64], y[:, 64:128]),
                    jnp.maximum(y[:, 128:192], y[:, 192:256]))
    fe_ref[...] = m * gg_ref[...] + gb_ref[...]


def _tc_head(xe4, p32, w1bd, g1t, b1t, w2bd, g2t, b2t, w3bd, gg, gb):
    nblocks = MPAD // 4 // BE4
    full = lambda shape: pl.BlockSpec(shape, lambda i: (0, 0))
    return pl.pallas_call(
        _tc_head_body,
        grid=(nblocks,),
        in_specs=[
            pl.BlockSpec((BE4, 32), lambda i: (i, 0)),
            pl.BlockSpec((BN, 32), lambda i: (i, 0)),
            full((32, 128)), full((1, 128)), full((1, 128)),
            full((128, 256)), full((1, 256)), full((1, 256)),
            full((256, 256)), full((1, 64)), full((1, 64)),
        ],
        out_specs=pl.BlockSpec((BN, C), lambda i: (i, 0)),
        out_shape=jax.ShapeDtypeStruct((NP, C), jnp.float32),
    )(xe4, p32, w1bd, g1t, b1t, w2bd, g2t, b2t, w3bd, gg, gb)


BM = 2048


def _tc_mlp_body(x_ref, g_ref, rg_ref, rb_ref, wa_ref, wb_ref,
                 fe_ref, h_ref):
    fe = x_ref[...] + g_ref[...].astype(jnp.float32)
    fe_ref[...] = fe
    x = fe * rg_ref[...] + rb_ref[...]
    y = jnp.dot(x, wa_ref[...], preferred_element_type=jnp.float32)
    y = jax.nn.gelu(y)
    h = jnp.dot(y, wb_ref[...], preferred_element_type=jnp.float32)
    h_ref[...] = h.astype(jnp.bfloat16)


def _tc_mlp(fe, g, rg1, rb1, wa, wb):
    nblocks = NP // BM
    full = lambda shape: pl.BlockSpec(shape, lambda i: (0, 0))
    return pl.pallas_call(
        _tc_mlp_body,
        grid=(nblocks,),
        in_specs=[
            pl.BlockSpec((BM, C), lambda i: (i, 0)),
            pl.BlockSpec((BM, C), lambda i: (i, 0)),
            full((1, C)), full((1, C)),
            full((C, 128)), full((128, C)),
        ],
        out_specs=[
            pl.BlockSpec((BM, C), lambda i: (i, 0)),
            pl.BlockSpec((BM, C), lambda i: (i, 0)),
        ],
        out_shape=[
            jax.ShapeDtypeStruct((NP, C), jnp.float32),
            jax.ShapeDtypeStruct((NP, C), jnp.bfloat16),
        ],
    )(fe, g, rg1, rb1, wa, wb)


def _tc_add_body(x_ref, g_ref, o_ref):
    o_ref[...] = x_ref[...] + g_ref[...].astype(jnp.float32)


def _tc_add(fe, g):
    nblocks = NP // BM
    return pl.pallas_call(
        _tc_add_body,
        grid=(nblocks,),
        in_specs=[
            pl.BlockSpec((BM, C), lambda i: (i, 0)),
            pl.BlockSpec((BM, C), lambda i: (i, 0)),
        ],
        out_specs=pl.BlockSpec((BM, C), lambda i: (i, 0)),
        out_shape=jax.ShapeDtypeStruct((NP, C), jnp.float32),
    )(fe, g)


def _block_diag(*ms):
    rows = sum(m.shape[0] for m in ms)
    cols = sum(m.shape[1] for m in ms)
    out = jnp.zeros((rows, cols), ms[0].dtype)
    r = c = 0
    for m in ms:
        out = out.at[r:r + m.shape[0], c:c + m.shape[1]].set(m)
        r += m.shape[0]
        c += m.shape[1]
    return out


def kernel(p, p_gs, f, group_idx, W1, g1, b1, W2, g2, b2, W3,
           gnb_g, gnb_b, Wa, Wb, rg, rb):
    del p_gs
    idx32 = group_idx.astype(jnp.int32)
    idx_flat = jnp.concatenate(
        [idx32.reshape(-1), jnp.zeros((MPAD - M,), jnp.int32)])

    pf = jnp.concatenate([p, f, jnp.zeros((N, 1), jnp.float32)], axis=1)
    ps8 = jnp.concatenate([p, jnp.zeros((N, 5), jnp.float32)], axis=1)
    ps8 = jnp.concatenate([ps8, jnp.zeros((NP - N, 8), jnp.float32)], axis=0)
    p32 = jnp.tile(ps8, (1, 4))

    w1p = jnp.concatenate([W1, jnp.zeros((1, 32), jnp.float32)], axis=0)
    w1bd = _block_diag(w1p, w1p, w1p, w1p)
    g1t = jnp.tile(g1, 4)[None, :]
    b1t = jnp.tile(b1, 4)[None, :]
    w2bd = _block_diag(W2, W2, W2, W2)
    g2t = jnp.tile(g2, 4)[None, :]
    b2t = jnp.tile(b2, 4)[None, :]
    w3bd = _block_diag(W3, W3, W3, W3)

    xe = _sc_head_gather(idx_flat, pf)
    fe = _tc_head(xe.reshape(MPAD // 4, 32), p32,
                  w1bd, g1t, b1t, w2bd, g2t, b2t, w3bd,
                  gnb_g[None, :], gnb_b[None, :])

    g = jnp.zeros((NP, C), jnp.bfloat16)
    for i in range(NBLK):
        fe, h = _tc_mlp(fe, g, rg[i][None, :], rb[i][None, :], Wa[i], Wb[i])
        g = _sc_gathermax(idx_flat, h)

    return _tc_add(fe, g)[:N]


# speedup vs baseline: 1.1595x; 1.1490x over previous
import functools

import jax
import jax.numpy as jnp
from jax import lax
from jax.experimental import pallas as pl
from jax.experimental.pallas import tpu as pltpu
from jax.experimental.pallas import tpu_sc as plsc

N = 50000
K = 16
CIN = 4
C = 64
H = 64
NBLK = 4

NW = 32
NP = 51200
M = N * K
MPAD = NP * K
CE = 2560
E0 = 48640
E1 = 2560
CH = 80
F0 = 3120
F1 = 80
KCH = K * CH

@functools.lru_cache(maxsize=None)
def _sc_kernels():
    mesh = plsc.VectorSubcoreMesh(core_axis_name="c", subcore_axis_name="s")

    @functools.partial(
        pl.kernel,
        out_type=jax.ShapeDtypeStruct((MPAD, 8), jnp.float32),
        mesh=mesh,
        scratch_types=[
            pltpu.VMEM((2, CE), jnp.int32),
            pltpu.VMEM((2, CE, 8), jnp.float32),
            pltpu.SemaphoreType.DMA,
            pltpu.SemaphoreType.DMA,
            pltpu.SemaphoreType.DMA,
        ],
        compiler_params=pltpu.CompilerParams(use_tc_tiling_on_sc=False),
    )
    def sc_head_gather(idx_hbm, pf_hbm, out_hbm, idx_v, buf_v, gsem, isem, osem):
        cid = lax.axis_index("c")
        sid = lax.axis_index("s")
        base = jnp.where(cid == 0, sid * E0, 16 * E0 + sid * E1)
        nch = jnp.where(cid == 0, E0 // CE, E1 // CE)

        def islice(c):
            return idx_hbm.at[pl.ds(base + c * CE, CE)]

        def oslice(c):
            return out_hbm.at[pl.ds(base + c * CE, CE)]

        @pl.when(nch >= 1)
        def _():
            pltpu.sync_copy(islice(0), idx_v.at[0])
            pltpu.async_copy(pf_hbm.at[idx_v.at[0]], buf_v.at[0], gsem)

        @pl.when(nch >= 2)
        def _():
            pltpu.async_copy(islice(1), idx_v.at[1], isem)

        @pl.loop(0, nch)
        def _chunk(c):
            p = c % 2
            pltpu.make_async_copy(pf_hbm.at[idx_v.at[p]], buf_v.at[p],
                                  gsem).wait()

            @pl.when(c >= 1)
            def _():
                pltpu.make_async_copy(buf_v.at[1 - p], oslice(c - 1),
                                      osem).wait()

            @pl.when(c + 1 < nch)
            def _():
                pltpu.make_async_copy(islice(c + 1), idx_v.at[1 - p],
                                      isem).wait()
                pltpu.async_copy(pf_hbm.at[idx_v.at[1 - p]], buf_v.at[1 - p],
                                 gsem)

            @pl.when(c + 2 < nch)
            def _():
                pltpu.async_copy(islice(c + 2), idx_v.at[p], isem)

            pltpu.async_copy(buf_v.at[p], oslice(c), osem)

        @pl.when(nch >= 1)
        def _():
            pltpu.make_async_copy(buf_v.at[(nch - 1) % 2], oslice(nch - 1),
                                  osem).wait()

    @functools.partial(
        pl.kernel,
        out_type=jax.ShapeDtypeStruct((NP, C), jnp.bfloat16),
        mesh=mesh,
        scratch_types=[
            pltpu.VMEM((2, KCH), jnp.int32),
            pltpu.VMEM((2, KCH, C), jnp.bfloat16),
            pltpu.VMEM((2, CH, C), jnp.bfloat16),
            pltpu.SemaphoreType.DMA,
            pltpu.SemaphoreType.DMA,
            pltpu.SemaphoreType.DMA,
        ],
        compiler_params=pltpu.CompilerParams(use_tc_tiling_on_sc=False),
    )
    def sc_gathermax(idx_hbm, h_hbm, out_hbm, idx_v, buf_v, gm_v,
                     gsem, isem, osem):
        cid = lax.axis_index("c")
        sid = lax.axis_index("s")
        nbase = jnp.where(cid == 0, sid * F0, 16 * F0 + sid * F1)
        nch = jnp.where(cid == 0, F0 // CH, F1 // CH)

        def islice(c):
            return idx_hbm.at[pl.ds((nbase + c * CH) * K, KCH)]

        def oslice(c):
            return out_hbm.at[pl.ds(nbase + c * CH, CH)]

        @pl.when(nch >= 1)
        def _():
            pltpu.sync_copy(islice(0), idx_v.at[0])
            pltpu.async_copy(h_hbm.at[idx_v.at[0]], buf_v.at[0], gsem)

        @pl.when(nch >= 2)
        def _():
            pltpu.async_copy(islice(1), idx_v.at[1], isem)

        @pl.loop(0, nch)
        def _chunk(c):
            p = c % 2
            pltpu.make_async_copy(h_hbm.at[idx_v.at[p]], buf_v.at[p],
                                  gsem).wait()

            @pl.when(c + 1 < nch)
            def _():
                pltpu.make_async_copy(islice(c + 1), idx_v.at[1 - p],
                                      isem).wait()
                pltpu.async_copy(h_hbm.at[idx_v.at[1 - p]], buf_v.at[1 - p],
                                 gsem)

            @pl.when(c + 2 < nch)
            def _():
                pltpu.async_copy(islice(c + 2), idx_v.at[p], isem)

            @pl.when(c >= 1)
            def _():
                pltpu.make_async_copy(gm_v.at[1 - p], oslice(c - 1),
                                      osem).wait()

            def body(r, _):
                for cc in range(C // 32):
                    sl = pl.ds(cc * 32, 32)
                    v = buf_v[p, r * K, sl]
                    for k in range(1, K):
                        v = jnp.maximum(v, buf_v[p, r * K + k, sl])
                    gm_v[p, r, sl] = v
                return 0

            lax.fori_loop(0, CH, body, 0)
            pltpu.async_copy(gm_v.at[p], oslice(c), osem)

        @pl.when(nch >= 1)
        def _():
            pltpu.make_async_copy(gm_v.at[(nch - 1) % 2], oslice(nch - 1),
                                  osem).wait()

    return sc_head_gather, sc_gathermax


def _sc_head_gather(idx_flat, pf):
    return _sc_kernels()[0](idx_flat, pf)


def _sc_gathermax(idx_flat, h):
    return _sc_kernels()[1](idx_flat, h)


BN = 2048
BE4 = BN * 4


def _tc_head_body(xe_ref, p32_ref, w1_ref, g1_ref, b1_ref, w2_ref, g2_ref,
                  b2_ref, w3_ref, gg_ref, gb_ref, fe_ref):
    a = jnp.dot(xe_ref[...], w1_ref[...],
                preferred_element_type=jnp.float32)
    b = jnp.dot(p32_ref[...], w1_ref[...],
                preferred_element_type=jnp.float32)
    b4 = jnp.broadcast_to(b[:, None, :], (BN, 4, 128)).reshape(BE4, 128)
    y = jax.nn.gelu((a - b4) * g1_ref[...] + b1_ref[...])
    y = jnp.dot(y, w2_ref[...], preferred_element_type=jnp.float32)
    y = jax.nn.gelu(y * g2_ref[...] + b2_ref[...])
    y = jnp.dot(y, w3_ref[...], preferred_element_type=jnp.float32)
    y = jnp.max(y.reshape(BN, 4, 256), axis=1)
    m = jnp.maximum(jnp.maximum(y[:, 0:64], y[:, 64:128]),
                    jnp.maximum(y[:, 128:192], y[:, 192:256]))
    fe_ref[...] = m * gg_ref[...] + gb_ref[...]


def _tc_head(xe4, p32, w1bd, g1t, b1t, w2bd, g2t, b2t, w3bd, gg, gb):
    nblocks = MPAD // 4 // BE4
    full = lambda shape: pl.BlockSpec(shape, lambda i: (0, 0))
    return pl.pallas_call(
        _tc_head_body,
        grid=(nblocks,),
        in_specs=[
            pl.BlockSpec((BE4, 32), lambda i: (i, 0)),
            pl.BlockSpec((BN, 32), lambda i: (i, 0)),
            full((32, 128)), full((1, 128)), full((1, 128)),
            full((128, 256)), full((1, 256)), full((1, 256)),
            full((256, 256)), full((1, 64)), full((1, 64)),
        ],
        out_specs=pl.BlockSpec((BN, C), lambda i: (i, 0)),
        out_shape=jax.ShapeDtypeStruct((NP, C), jnp.float32),
    )(xe4, p32, w1bd, g1t, b1t, w2bd, g2t, b2t, w3bd, gg, gb)


BM = 2048


def _tc_mlp_body(x_ref, g_ref, rg_ref, rb_ref, wa_ref, wb_ref,
                 fe_ref, h_ref):
    fe = x_ref[...] + g_ref[...].astype(jnp.float32)
    fe_ref[...] = fe
    x = fe * rg_ref[...] + rb_ref[...]
    y = jnp.dot(x, wa_ref[...], preferred_element_type=jnp.float32)
    y = jax.nn.gelu(y)
    h = jnp.dot(y, wb_ref[...], preferred_element_type=jnp.float32)
    h_ref[...] = h.astype(jnp.bfloat16)


def _tc_mlp(fe, g, rg1, rb1, wa, wb):
    nblocks = NP // BM
    full = lambda shape: pl.BlockSpec(shape, lambda i: (0, 0))
    return pl.pallas_call(
        _tc_mlp_body,
        grid=(nblocks,),
        in_specs=[
            pl.BlockSpec((BM, C), lambda i: (i, 0)),
            pl.BlockSpec((BM, C), lambda i: (i, 0)),
            full((1, C)), full((1, C)),
            full((C, 128)), full((128, C)),
        ],
        out_specs=[
            pl.BlockSpec((BM, C), lambda i: (i, 0)),
            pl.BlockSpec((BM, C), lambda i: (i, 0)),
        ],
        out_shape=[
            jax.ShapeDtypeStruct((NP, C), jnp.float32),
            jax.ShapeDtypeStruct((NP, C), jnp.bfloat16),
        ],
    )(fe, g, rg1, rb1, wa, wb)


def _tc_add_body(x_ref, g_ref, o_ref):
    o_ref[...] = x_ref[...] + g_ref[...].astype(jnp.float32)


def _tc_add(fe, g):
    nblocks = NP // BM
    return pl.pallas_call(
        _tc_add_body,
        grid=(nblocks,),
        in_specs=[
            pl.BlockSpec((BM, C), lambda i: (i, 0)),
            pl.BlockSpec((BM, C), lambda i: (i, 0)),
        ],
        out_specs=pl.BlockSpec((BM, C), lambda i: (i, 0)),
        out_shape=jax.ShapeDtypeStruct((NP, C), jnp.float32),
    )(fe, g)


def _block_diag(*ms):
    rows = sum(m.shape[0] for m in ms)
    cols = sum(m.shape[1] for m in ms)
    out = jnp.zeros((rows, cols), ms[0].dtype)
    r = c = 0
    for m in ms:
        out = out.at[r:r + m.shape[0], c:c + m.shape[1]].set(m)
        r += m.shape[0]
        c += m.shape[1]
    return out


def kernel(p, p_gs, f, group_idx, W1, g1, b1, W2, g2, b2, W3,
           gnb_g, gnb_b, Wa, Wb, rg, rb):
    del p_gs
    idx32 = group_idx.astype(jnp.int32)
    idx_flat = jnp.concatenate(
        [idx32.reshape(-1), jnp.zeros((MPAD - M,), jnp.int32)])

    pf = jnp.concatenate([p, f, jnp.zeros((N, 1), jnp.float32)], axis=1)
    ps8 = jnp.concatenate([p, jnp.zeros((N, 5), jnp.float32)], axis=1)
    ps8 = jnp.concatenate([ps8, jnp.zeros((NP - N, 8), jnp.float32)], axis=0)
    p32 = jnp.tile(ps8, (1, 4))

    w1p = jnp.concatenate([W1, jnp.zeros((1, 32), jnp.float32)], axis=0)
    w1bd = _block_diag(w1p, w1p, w1p, w1p)
    g1t = jnp.tile(g1, 4)[None, :]
    b1t = jnp.tile(b1, 4)[None, :]
    w2bd = _block_diag(W2, W2, W2, W2)
    g2t = jnp.tile(g2, 4)[None, :]
    b2t = jnp.tile(b2, 4)[None, :]
    w3bd = _block_diag(W3, W3, W3, W3)

    xe = _sc_head_gather(idx_flat, pf)
    fe = _tc_head(xe.reshape(MPAD // 4, 32), p32,
                  w1bd, g1t, b1t, w2bd, g2t, b2t, w3bd,
                  gnb_g[None, :], gnb_b[None, :])

    g = jnp.zeros((NP, C), jnp.bfloat16)
    for i in range(NBLK):
        fe, h = _tc_mlp(fe, g, rg[i][None, :], rb[i][None, :], Wa[i], Wb[i])
        g = _sc_gathermax(idx_flat, h)

    return _tc_add(fe, g)[:N]
